# Initial kernel scaffold; baseline (speedup 1.0000x reference)
import jax
import jax.numpy as jnp
from jax.experimental import pallas as pl


def kernel(x, edge_index, batch, gin0_W1, gin0_b1, gin0_W2, gin0_b2, gin1_W1, gin1_b1, gin1_W2, gin1_b2, gin2_W1, gin2_b1, gin2_W2, gin2_b2, lin1_W, lin1_b, lin2_W, lin2_b):
    raise NotImplementedError("write your pallas kernel here")


# speedup vs baseline: 5.4455x; 5.4455x over previous
import functools

import jax
import jax.numpy as jnp
from jax import lax
from jax.experimental import pallas as pl
from jax.experimental.pallas import tpu as pltpu
from jax.experimental.pallas import tpu_sc as plsc

N = 10000
E = 320000
D0 = 128
H = 64
C = 2
G = 64

NC = 2
NS = 16
NW = NC * NS
EPW = E // NW
CHUNK = 80
NCHUNK = EPW // CHUNK
RPT = 624
RPT_LAST = N - RPT * (NS - 1)


@functools.lru_cache(maxsize=None)
def _make_aggr(d):
    mesh = plsc.VectorSubcoreMesh(core_axis_name="c", subcore_axis_name="s")

    @functools.partial(
        pl.kernel,
        out_type=jax.ShapeDtypeStruct((NC, N, d), jnp.float32),
        mesh=mesh,
        scratch_types=[
            pltpu.VMEM((CHUNK,), jnp.int32),
            pltpu.VMEM((CHUNK,), jnp.int32),
            pltpu.VMEM((CHUNK, d), jnp.float32),
            pltpu.VMEM_SHARED((N, d), jnp.float32),
            pltpu.SemaphoreType.DMA,
        ],
        compiler_params=pltpu.CompilerParams(use_tc_tiling_on_sc=False),
    )
    def aggr(h_hbm, src_hbm, dst_hbm, out_hbm, sidx, didx, rows, acc, sem):
        cid = lax.axis_index("c")
        sid = lax.axis_index("s")
        wid = cid * NS + sid
        r0 = sid * RPT
        @pl.when(sid < NS - 1)
        def _():
            pltpu.sync_copy(h_hbm.at[pl.ds(r0, RPT)], acc.at[pl.ds(r0, RPT)])

        @pl.when(sid == NS - 1)
        def _():
            pltpu.sync_copy(h_hbm.at[pl.ds(RPT * (NS - 1), RPT_LAST)],
                            acc.at[pl.ds(RPT * (NS - 1), RPT_LAST)])

        plsc.subcore_barrier()

        ebase = wid * EPW

        def body(i, _):
            base = ebase + i * CHUNK
            pltpu.sync_copy(src_hbm.at[pl.ds(base, CHUNK)], sidx)
            pltpu.sync_copy(dst_hbm.at[pl.ds(base, CHUNK)], didx)
            pltpu.async_copy(h_hbm.at[sidx], rows, sem).wait()
            pltpu.sync_copy(rows, acc.at[didx], add=True)
            return ()

        lax.fori_loop(0, NCHUNK, body, ())
        plsc.subcore_barrier()

        @pl.when(sid < NS - 1)
        def _():
            pltpu.sync_copy(acc.at[pl.ds(r0, RPT)],
                            out_hbm.at[cid, pl.ds(r0, RPT)])

        @pl.when(sid == NS - 1)
        def _():
            pltpu.sync_copy(acc.at[pl.ds(RPT * (NS - 1), RPT_LAST)],
                            out_hbm.at[cid, pl.ds(RPT * (NS - 1), RPT_LAST)])

    return aggr


_BN = 2000
_NBLK = N // _BN


def _mlp_body(h_ref, a_ref, w1_ref, b1_ref, w2_ref, b2_ref, o_ref):
    acc = a_ref[0] + a_ref[1] - h_ref[...]
    t = jnp.dot(acc, w1_ref[...], preferred_element_type=jnp.float32)
    t = jnp.maximum(t + b1_ref[...], 0.0)
    u = jnp.dot(t, w2_ref[...], preferred_element_type=jnp.float32)
    o_ref[...] = jnp.maximum(u + b2_ref[...], 0.0)


def _make_mlp(d):
    return pl.pallas_call(
        _mlp_body,
        grid=(_NBLK,),
        in_specs=[
            pl.BlockSpec((_BN, d), lambda i: (i, 0)),
            pl.BlockSpec((NC, _BN, d), lambda i: (0, i, 0)),
            pl.BlockSpec((d, H), lambda i: (0, 0)),
            pl.BlockSpec((1, H), lambda i: (0, 0)),
            pl.BlockSpec((H, H), lambda i: (0, 0)),
            pl.BlockSpec((1, H), lambda i: (0, 0)),
        ],
        out_specs=pl.BlockSpec((_BN, H), lambda i: (i, 0)),
        out_shape=jax.ShapeDtypeStruct((N, H), jnp.float32),
    )


_MLP = {d: _make_mlp(d) for d in (D0, H)}


def _final_body(h_ref, a_ref, batch_ref, w1_ref, b1_ref, w2_ref, b2_ref,
                l1w_ref, l1b_ref, l2w_ref, l2b_ref, o_ref, sums, counts):
    i = pl.program_id(0)

    @pl.when(i == 0)
    def _():
        sums[...] = jnp.zeros_like(sums)
        counts[...] = jnp.zeros_like(counts)

    acc = a_ref[0] + a_ref[1] - h_ref[...]
    t = jnp.dot(acc, w1_ref[...], preferred_element_type=jnp.float32)
    t = jnp.maximum(t + b1_ref[...], 0.0)
    u = jnp.dot(t, w2_ref[...], preferred_element_type=jnp.float32)
    h3 = jnp.maximum(u + b2_ref[...], 0.0)

    b = batch_ref[0, 0, :]
    onehot = (b[:, None] == lax.broadcasted_iota(jnp.int32, (_BN, G), 1))
    onehot = onehot.astype(jnp.float32)
    sums[...] += lax.dot_general(onehot, h3, (((0,), (0,)), ((), ())),
                                 preferred_element_type=jnp.float32)
    counts[...] += lax.dot_general(onehot, jnp.ones((_BN, 1), jnp.float32),
                                   (((0,), (0,)), ((), ())),
                                   preferred_element_type=jnp.float32)

    @pl.when(i == _NBLK - 1)
    def _():
        pooled = sums[...] / jnp.maximum(counts[...], 1.0)
        y = jnp.dot(pooled, l1w_ref[...], preferred_element_type=jnp.float32)
        y = jnp.maximum(y + l1b_ref[...], 0.0)
        o_ref[...] = (jnp.dot(y, l2w_ref[...],
                              preferred_element_type=jnp.float32)
                      + l2b_ref[...])


_FINAL = pl.pallas_call(
    _final_body,
    grid=(_NBLK,),
    in_specs=[
        pl.BlockSpec((_BN, H), lambda i: (i, 0)),
        pl.BlockSpec((NC, _BN, H), lambda i: (0, i, 0)),
        pl.BlockSpec((1, 1, _BN), lambda i: (i, 0, 0)),
        pl.BlockSpec((H, H), lambda i: (0, 0)),
        pl.BlockSpec((1, H), lambda i: (0, 0)),
        pl.BlockSpec((H, H), lambda i: (0, 0)),
        pl.BlockSpec((1, H), lambda i: (0, 0)),
        pl.BlockSpec((H, H), lambda i: (0, 0)),
        pl.BlockSpec((1, H), lambda i: (0, 0)),
        pl.BlockSpec((H, C), lambda i: (0, 0)),
        pl.BlockSpec((1, C), lambda i: (0, 0)),
    ],
    out_specs=pl.BlockSpec((G, C), lambda i: (0, 0)),
    out_shape=jax.ShapeDtypeStruct((G, C), jnp.float32),
    scratch_shapes=[
        pltpu.VMEM((G, H), jnp.float32),
        pltpu.VMEM((G, 1), jnp.float32),
    ],
)


def kernel(x, edge_index, batch,
           gin0_W1, gin0_b1, gin0_W2, gin0_b2,
           gin1_W1, gin1_b1, gin1_W2, gin1_b2,
           gin2_W1, gin2_b1, gin2_W2, gin2_b2,
           lin1_W, lin1_b, lin2_W, lin2_b):
    src = edge_index[0]
    dst = edge_index[1]
    batch3 = batch.astype(jnp.int32).reshape(_NBLK, 1, _BN)

    a = _make_aggr(D0)(x, src, dst)
    h1 = _MLP[D0](x, a, gin0_W1, gin0_b1.reshape(1, H),
                  gin0_W2, gin0_b2.reshape(1, H))
    a = _make_aggr(H)(h1, src, dst)
    h2 = _MLP[H](h1, a, gin1_W1, gin1_b1.reshape(1, H),
                 gin1_W2, gin1_b2.reshape(1, H))
    a = _make_aggr(H)(h2, src, dst)
    out = _FINAL(h2, a, batch3, gin2_W1, gin2_b1.reshape(1, H),
                 gin2_W2, gin2_b2.reshape(1, H),
                 lin1_W, lin1_b.reshape(1, H),
                 lin2_W, lin2_b.reshape(1, C))
    return out


# speedup vs baseline: 13.6876x; 2.5135x over previous
import functools

import jax
import jax.numpy as jnp
from jax import lax
from jax.experimental import pallas as pl
from jax.experimental.pallas import tpu as pltpu
from jax.experimental.pallas import tpu_sc as plsc

N = 10000
E = 320000
D0 = 128
H = 64
C = 2
G = 64

NC = 2
NS = 16
NW = NC * NS
EPW = E // NW
CHUNK = 100
NCHUNK = EPW // CHUNK
NPAIR = NCHUNK // 2
RPT = 624
RPT_LAST = N - RPT * (NS - 1)


@functools.lru_cache(maxsize=None)
def _make_aggr(d):
    mesh = plsc.VectorSubcoreMesh(core_axis_name="c", subcore_axis_name="s")

    @functools.partial(
        pl.kernel,
        out_type=jax.ShapeDtypeStruct((NC, N, d), jnp.float32),
        mesh=mesh,
        scratch_types=[
            pltpu.VMEM((NCHUNK, CHUNK), jnp.int32),
            pltpu.VMEM((NCHUNK, CHUNK), jnp.int32),
            pltpu.VMEM((CHUNK, d), jnp.float32),
            pltpu.VMEM((CHUNK, d), jnp.float32),
            pltpu.VMEM_SHARED((N, d), jnp.float32),
            pltpu.SemaphoreType.DMA,
            pltpu.SemaphoreType.DMA,
        ],
        compiler_params=pltpu.CompilerParams(use_tc_tiling_on_sc=False),
    )
    def aggr(h_hbm, src_hbm, dst_hbm, out_hbm, sidx, didx, rows0, rows1,
             acc, sem0, sem1):
        cid = lax.axis_index("c")
        sid = lax.axis_index("s")
        wid = cid * NS + sid
        r0 = sid * RPT
        pltpu.sync_copy(src_hbm.at[pl.ds(wid * NCHUNK, NCHUNK)], sidx)
        pltpu.sync_copy(dst_hbm.at[pl.ds(wid * NCHUNK, NCHUNK)], didx)

        @pl.when(sid < NS - 1)
        def _():
            pltpu.sync_copy(h_hbm.at[pl.ds(r0, RPT)], acc.at[pl.ds(r0, RPT)])

        @pl.when(sid == NS - 1)
        def _():
            pltpu.sync_copy(h_hbm.at[pl.ds(RPT * (NS - 1), RPT_LAST)],
                            acc.at[pl.ds(RPT * (NS - 1), RPT_LAST)])

        plsc.subcore_barrier()

        pltpu.async_copy(h_hbm.at[sidx.at[0]], rows0, sem0)

        def body(j, _):
            pltpu.async_copy(h_hbm.at[sidx.at[2 * j + 1]], rows1, sem1)
            pltpu.make_async_copy(h_hbm.at[sidx.at[2 * j]], rows0,
                                  sem0).wait()
            pltpu.sync_copy(rows0, acc.at[didx.at[2 * j]], add=True)

            @pl.when(j < NPAIR - 1)
            def _():
                pltpu.async_copy(h_hbm.at[sidx.at[2 * j + 2]], rows0, sem0)

            pltpu.make_async_copy(h_hbm.at[sidx.at[2 * j + 1]], rows1,
                                  sem1).wait()
            pltpu.sync_copy(rows1, acc.at[didx.at[2 * j + 1]], add=True)
            return ()

        lax.fori_loop(0, NPAIR, body, ())
        plsc.subcore_barrier()

        @pl.when(sid < NS - 1)
        def _():
            pltpu.sync_copy(acc.at[pl.ds(r0, RPT)],
                            out_hbm.at[cid, pl.ds(r0, RPT)])

        @pl.when(sid == NS - 1)
        def _():
            pltpu.sync_copy(acc.at[pl.ds(RPT * (NS - 1), RPT_LAST)],
                            out_hbm.at[cid, pl.ds(RPT * (NS - 1), RPT_LAST)])

    return aggr


_BN = 2000
_NBLK = N // _BN


def _mlp_body(h_ref, a_ref, w1_ref, b1_ref, w2_ref, b2_ref, o_ref):
    acc = a_ref[0] + a_ref[1] - h_ref[...]
    t = jnp.dot(acc, w1_ref[...], preferred_element_type=jnp.float32)
    t = jnp.maximum(t + b1_ref[...], 0.0)
    u = jnp.dot(t, w2_ref[...], preferred_element_type=jnp.float32)
    o_ref[...] = jnp.maximum(u + b2_ref[...], 0.0)


def _make_mlp(d):
    return pl.pallas_call(
        _mlp_body,
        grid=(_NBLK,),
        in_specs=[
            pl.BlockSpec((_BN, d), lambda i: (i, 0)),
            pl.BlockSpec((NC, _BN, d), lambda i: (0, i, 0)),
            pl.BlockSpec((d, H), lambda i: (0, 0)),
            pl.BlockSpec((1, H), lambda i: (0, 0)),
            pl.BlockSpec((H, H), lambda i: (0, 0)),
            pl.BlockSpec((1, H), lambda i: (0, 0)),
        ],
        out_specs=pl.BlockSpec((_BN, H), lambda i: (i, 0)),
        out_shape=jax.ShapeDtypeStruct((N, H), jnp.float32),
    )


_MLP = {d: _make_mlp(d) for d in (D0, H)}


def _final_body(h_ref, a_ref, batch_ref, w1_ref, b1_ref, w2_ref, b2_ref,
                l1w_ref, l1b_ref, l2w_ref, l2b_ref, o_ref, sums, counts):
    i = pl.program_id(0)

    @pl.when(i == 0)
    def _():
        sums[...] = jnp.zeros_like(sums)
        counts[...] = jnp.zeros_like(counts)

    acc = a_ref[0] + a_ref[1] - h_ref[...]
    t = jnp.dot(acc, w1_ref[...], preferred_element_type=jnp.float32)
    t = jnp.maximum(t + b1_ref[...], 0.0)
    u = jnp.dot(t, w2_ref[...], preferred_element_type=jnp.float32)
    h3 = jnp.maximum(u + b2_ref[...], 0.0)

    b = batch_ref[0, 0, :]
    onehot = (b[:, None] == lax.broadcasted_iota(jnp.int32, (_BN, G), 1))
    onehot = onehot.astype(jnp.float32)
    sums[...] += lax.dot_general(onehot, h3, (((0,), (0,)), ((), ())),
                                 preferred_element_type=jnp.float32)
    counts[...] += lax.dot_general(onehot, jnp.ones((_BN, 1), jnp.float32),
                                   (((0,), (0,)), ((), ())),
                                   preferred_element_type=jnp.float32)

    @pl.when(i == _NBLK - 1)
    def _():
        pooled = sums[...] / jnp.maximum(counts[...], 1.0)
        y = jnp.dot(pooled, l1w_ref[...], preferred_element_type=jnp.float32)
        y = jnp.maximum(y + l1b_ref[...], 0.0)
        o_ref[...] = (jnp.dot(y, l2w_ref[...],
                              preferred_element_type=jnp.float32)
                      + l2b_ref[...])


_FINAL = pl.pallas_call(
    _final_body,
    grid=(_NBLK,),
    in_specs=[
        pl.BlockSpec((_BN, H), lambda i: (i, 0)),
        pl.BlockSpec((NC, _BN, H), lambda i: (0, i, 0)),
        pl.BlockSpec((1, 1, _BN), lambda i: (i, 0, 0)),
        pl.BlockSpec((H, H), lambda i: (0, 0)),
        pl.BlockSpec((1, H), lambda i: (0, 0)),
        pl.BlockSpec((H, H), lambda i: (0, 0)),
        pl.BlockSpec((1, H), lambda i: (0, 0)),
        pl.BlockSpec((H, H), lambda i: (0, 0)),
        pl.BlockSpec((1, H), lambda i: (0, 0)),
        pl.BlockSpec((H, C), lambda i: (0, 0)),
        pl.BlockSpec((1, C), lambda i: (0, 0)),
    ],
    out_specs=pl.BlockSpec((G, C), lambda i: (0, 0)),
    out_shape=jax.ShapeDtypeStruct((G, C), jnp.float32),
    scratch_shapes=[
        pltpu.VMEM((G, H), jnp.float32),
        pltpu.VMEM((G, 1), jnp.float32),
    ],
)


def kernel(x, edge_index, batch,
           gin0_W1, gin0_b1, gin0_W2, gin0_b2,
           gin1_W1, gin1_b1, gin1_W2, gin1_b2,
           gin2_W1, gin2_b1, gin2_W2, gin2_b2,
           lin1_W, lin1_b, lin2_W, lin2_b):
    src = edge_index[0].reshape(E // CHUNK, CHUNK)
    dst = edge_index[1].reshape(E // CHUNK, CHUNK)
    batch3 = batch.astype(jnp.int32).reshape(_NBLK, 1, _BN)

    a = _make_aggr(D0)(x, src, dst)
    h1 = _MLP[D0](x, a, gin0_W1, gin0_b1.reshape(1, H),
                  gin0_W2, gin0_b2.reshape(1, H))
    a = _make_aggr(H)(h1, src, dst)
    h2 = _MLP[H](h1, a, gin1_W1, gin1_b1.reshape(1, H),
                 gin1_W2, gin1_b2.reshape(1, H))
    a = _make_aggr(H)(h2, src, dst)
    out = _FINAL(h2, a, batch3, gin2_W1, gin2_b1.reshape(1, H),
                 gin2_W2, gin2_b2.reshape(1, H),
                 lin1_W, lin1_b.reshape(1, H),
                 lin2_W, lin2_b.reshape(1, C))
    return out


# speedup vs baseline: 13.9546x; 1.0195x over previous
import functools

import jax
import jax.numpy as jnp
from jax import lax
from jax.experimental import pallas as pl
from jax.experimental.pallas import tpu as pltpu
from jax.experimental.pallas import tpu_sc as plsc

N = 10000
E = 320000
D0 = 128
H = 64
C = 2
G = 64

NC = 2
NS = 16
NW = NC * NS
CHUNK = 125
NBUF = 4

EPW = E // NW
NCHUNK = EPW // CHUNK
NITER = NCHUNK // NBUF

EPT = E // NS
NCHUNK0 = EPT // CHUNK
NITER0 = NCHUNK0 // NBUF

RPT = 624
RPT_LAST = N - RPT * (NS - 1)

_SC_PARAMS = pltpu.CompilerParams(use_tc_tiling_on_sc=False)


def _stripe_copy(src, dst, sid):
    r0 = sid * RPT

    @pl.when(sid < NS - 1)
    def _():
        pltpu.sync_copy(src.at[pl.ds(r0, RPT)], dst.at[pl.ds(r0, RPT)])

    @pl.when(sid == NS - 1)
    def _():
        pltpu.sync_copy(src.at[pl.ds(RPT * (NS - 1), RPT_LAST)],
                        dst.at[pl.ds(RPT * (NS - 1), RPT_LAST)])


def _edge_loop(h_ref, sidx, didx, rows, acc, gsem, ssem, niter):
    for b in range(NBUF):
        pltpu.async_copy(h_ref.at[sidx.at[b]], rows[b], gsem[b])

    def body(j, _):
        k0 = j * NBUF
        for b in range(NBUF):
            pltpu.make_async_copy(h_ref.at[sidx.at[k0 + b]], rows[b],
                                  gsem[b]).wait()
            pltpu.async_copy(rows[b], acc.at[didx.at[k0 + b]], ssem[b],
                             add=True)
        for b in range(NBUF):
            pltpu.make_async_copy(rows[b], acc.at[didx.at[k0 + b]],
                                  ssem[b]).wait()

            @pl.when(j < niter - 1)
            def _():
                pltpu.async_copy(h_ref.at[sidx.at[k0 + NBUF + b]], rows[b],
                                 gsem[b])
        return ()

    lax.fori_loop(0, niter, body, ())


@functools.lru_cache(maxsize=None)
def _make_aggr0():
    mesh = plsc.VectorSubcoreMesh(core_axis_name="c", subcore_axis_name="s")

    @functools.partial(
        pl.kernel,
        out_type=jax.ShapeDtypeStruct((NC, N, H), jnp.float32),
        mesh=mesh,
        scratch_types=[
            pltpu.VMEM((NCHUNK0, CHUNK), jnp.int32),
            pltpu.VMEM((NCHUNK0, CHUNK), jnp.int32),
            [pltpu.VMEM((CHUNK, H), jnp.float32) for _ in range(NBUF)],
            pltpu.VMEM_SHARED((N, H), jnp.float32),
            [pltpu.SemaphoreType.DMA for _ in range(NBUF)],
            [pltpu.SemaphoreType.DMA for _ in range(NBUF)],
        ],
        compiler_params=_SC_PARAMS,
    )
    def aggr0(hlo_hbm, hhi_hbm, src_hbm, dst_hbm, out_hbm, sidx, didx, rows,
              acc, gsem, ssem):
        cid = lax.axis_index("c")
        sid = lax.axis_index("s")
        pltpu.sync_copy(src_hbm.at[pl.ds(sid * NCHUNK0, NCHUNK0)], sidx)
        pltpu.sync_copy(dst_hbm.at[pl.ds(sid * NCHUNK0, NCHUNK0)], didx)

        @pl.when(cid == 0)
        def _():
            _stripe_copy(hlo_hbm, acc, sid)

        @pl.when(cid == 1)
        def _():
            _stripe_copy(hhi_hbm, acc, sid)

        plsc.subcore_barrier()

        @pl.when(cid == 0)
        def _():
            _edge_loop(hlo_hbm, sidx, didx, rows, acc, gsem, ssem, NITER0)

        @pl.when(cid == 1)
        def _():
            _edge_loop(hhi_hbm, sidx, didx, rows, acc, gsem, ssem, NITER0)

        plsc.subcore_barrier()
        _stripe_copy(acc, out_hbm.at[cid], sid)

    return aggr0


@functools.lru_cache(maxsize=None)
def _make_aggr():
    mesh = plsc.VectorSubcoreMesh(core_axis_name="c", subcore_axis_name="s")

    @functools.partial(
        pl.kernel,
        out_type=jax.ShapeDtypeStruct((NC, N, H), jnp.float32),
        mesh=mesh,
        scratch_types=[
            pltpu.VMEM((NCHUNK, CHUNK), jnp.int32),
            pltpu.VMEM((NCHUNK, CHUNK), jnp.int32),
            [pltpu.VMEM((CHUNK, H), jnp.float32) for _ in range(NBUF)],
            pltpu.VMEM_SHARED((N, H), jnp.float32),
            [pltpu.SemaphoreType.DMA for _ in range(NBUF)],
            [pltpu.SemaphoreType.DMA for _ in range(NBUF)],
        ],
        compiler_params=_SC_PARAMS,
    )
    def aggr(h_hbm, src_hbm, dst_hbm, out_hbm, sidx, didx, rows,
             acc, gsem, ssem):
        cid = lax.axis_index("c")
        sid = lax.axis_index("s")
        wid = cid * NS + sid
        pltpu.sync_copy(src_hbm.at[pl.ds(wid * NCHUNK, NCHUNK)], sidx)
        pltpu.sync_copy(dst_hbm.at[pl.ds(wid * NCHUNK, NCHUNK)], didx)
        _stripe_copy(h_hbm, acc, sid)
        plsc.subcore_barrier()
        _edge_loop(h_hbm, sidx, didx, rows, acc, gsem, ssem, NITER)
        plsc.subcore_barrier()
        _stripe_copy(acc, out_hbm.at[cid], sid)

    return aggr


_BN = 2000
_NBLK = N // _BN


def _mlp0_body(a_ref, w1_ref, b1_ref, w2_ref, b2_ref, o_ref):
    t = (jnp.dot(a_ref[0], w1_ref[:H], preferred_element_type=jnp.float32)
         + jnp.dot(a_ref[1], w1_ref[H:], preferred_element_type=jnp.float32))
    t = jnp.maximum(t + b1_ref[...], 0.0)
    u = jnp.dot(t, w2_ref[...], preferred_element_type=jnp.float32)
    o_ref[...] = jnp.maximum(u + b2_ref[...], 0.0)


_MLP0 = pl.pallas_call(
    _mlp0_body,
    grid=(_NBLK,),
    in_specs=[
        pl.BlockSpec((NC, _BN, H), lambda i: (0, i, 0)),
        pl.BlockSpec((D0, H), lambda i: (0, 0)),
        pl.BlockSpec((1, H), lambda i: (0, 0)),
        pl.BlockSpec((H, H), lambda i: (0, 0)),
        pl.BlockSpec((1, H), lambda i: (0, 0)),
    ],
    out_specs=pl.BlockSpec((_BN, H), lambda i: (i, 0)),
    out_shape=jax.ShapeDtypeStruct((N, H), jnp.float32),
)


def _mlp_body(h_ref, a_ref, w1_ref, b1_ref, w2_ref, b2_ref, o_ref):
    acc = a_ref[0] + a_ref[1] - h_ref[...]
    t = jnp.dot(acc, w1_ref[...], preferred_element_type=jnp.float32)
    t = jnp.maximum(t + b1_ref[...], 0.0)
    u = jnp.dot(t, w2_ref[...], preferred_element_type=jnp.float32)
    o_ref[...] = jnp.maximum(u + b2_ref[...], 0.0)


_MLP = pl.pallas_call(
    _mlp_body,
    grid=(_NBLK,),
    in_specs=[
        pl.BlockSpec((_BN, H), lambda i: (i, 0)),
        pl.BlockSpec((NC, _BN, H), lambda i: (0, i, 0)),
        pl.BlockSpec((H, H), lambda i: (0, 0)),
        pl.BlockSpec((1, H), lambda i: (0, 0)),
        pl.BlockSpec((H, H), lambda i: (0, 0)),
        pl.BlockSpec((1, H), lambda i: (0, 0)),
    ],
    out_specs=pl.BlockSpec((_BN, H), lambda i: (i, 0)),
    out_shape=jax.ShapeDtypeStruct((N, H), jnp.float32),
)


def _final_body(h_ref, a_ref, batch_ref, w1_ref, b1_ref, w2_ref, b2_ref,
                l1w_ref, l1b_ref, l2w_ref, l2b_ref, o_ref, sums, counts):
    i = pl.program_id(0)

    @pl.when(i == 0)
    def _():
        sums[...] = jnp.zeros_like(sums)
        counts[...] = jnp.zeros_like(counts)

    acc = a_ref[0] + a_ref[1] - h_ref[...]
    t = jnp.dot(acc, w1_ref[...], preferred_element_type=jnp.float32)
    t = jnp.maximum(t + b1_ref[...], 0.0)
    u = jnp.dot(t, w2_ref[...], preferred_element_type=jnp.float32)
    h3 = jnp.maximum(u + b2_ref[...], 0.0)

    b = batch_ref[0, 0, :]
    onehot = (b[:, None] == lax.broadcasted_iota(jnp.int32, (_BN, G), 1))
    onehot = onehot.astype(jnp.float32)
    sums[...] += lax.dot_general(onehot, h3, (((0,), (0,)), ((), ())),
                                 preferred_element_type=jnp.float32)
    counts[...] += lax.dot_general(onehot, jnp.ones((_BN, 1), jnp.float32),
                                   (((0,), (0,)), ((), ())),
                                   preferred_element_type=jnp.float32)

    @pl.when(i == _NBLK - 1)
    def _():
        pooled = sums[...] / jnp.maximum(counts[...], 1.0)
        y = jnp.dot(pooled, l1w_ref[...], preferred_element_type=jnp.float32)
        y = jnp.maximum(y + l1b_ref[...], 0.0)
        o_ref[...] = (jnp.dot(y, l2w_ref[...],
                              preferred_element_type=jnp.float32)
                      + l2b_ref[...])


_FINAL = pl.pallas_call(
    _final_body,
    grid=(_NBLK,),
    in_specs=[
        pl.BlockSpec((_BN, H), lambda i: (i, 0)),
        pl.BlockSpec((NC, _BN, H), lambda i: (0, i, 0)),
        pl.BlockSpec((1, 1, _BN), lambda i: (i, 0, 0)),
        pl.BlockSpec((H, H), lambda i: (0, 0)),
        pl.BlockSpec((1, H), lambda i: (0, 0)),
        pl.BlockSpec((H, H), lambda i: (0, 0)),
        pl.BlockSpec((1, H), lambda i: (0, 0)),
        pl.BlockSpec((H, H), lambda i: (0, 0)),
        pl.BlockSpec((1, H), lambda i: (0, 0)),
        pl.BlockSpec((H, C), lambda i: (0, 0)),
        pl.BlockSpec((1, C), lambda i: (0, 0)),
    ],
    out_specs=pl.BlockSpec((G, C), lambda i: (0, 0)),
    out_shape=jax.ShapeDtypeStruct((G, C), jnp.float32),
    scratch_shapes=[
        pltpu.VMEM((G, H), jnp.float32),
        pltpu.VMEM((G, 1), jnp.float32),
    ],
)


def kernel(x, edge_index, batch,
           gin0_W1, gin0_b1, gin0_W2, gin0_b2,
           gin1_W1, gin1_b1, gin1_W2, gin1_b2,
           gin2_W1, gin2_b1, gin2_W2, gin2_b2,
           lin1_W, lin1_b, lin2_W, lin2_b):
    src = edge_index[0].reshape(E // CHUNK, CHUNK)
    dst = edge_index[1].reshape(E // CHUNK, CHUNK)
    batch3 = batch.astype(jnp.int32).reshape(_NBLK, 1, _BN)

    a = _make_aggr0()(x[:, :H], x[:, H:], src, dst)
    h1 = _MLP0(a, gin0_W1, gin0_b1.reshape(1, H),
               gin0_W2, gin0_b2.reshape(1, H))
    a = _make_aggr()(h1, src, dst)
    h2 = _MLP(h1, a, gin1_W1, gin1_b1.reshape(1, H),
              gin1_W2, gin1_b2.reshape(1, H))
    a = _make_aggr()(h2, src, dst)
    out = _FINAL(h2, a, batch3, gin2_W1, gin2_b1.reshape(1, H),
                 gin2_W2, gin2_b2.reshape(1, H),
                 lin1_W, lin1_b.reshape(1, H),
                 lin2_W, lin2_b.reshape(1, C))
    return out


# speedup vs baseline: 16.1748x; 1.1591x over previous
import functools

import jax
import jax.numpy as jnp
from jax import lax
from jax.experimental import pallas as pl
from jax.experimental.pallas import tpu as pltpu
from jax.experimental.pallas import tpu_sc as plsc

N = 10000
E = 320000
D0 = 128
H = 64
C = 2
G = 64

NC = 2
NS = 16
NW = NC * NS
CHUNK = 128
NBUF = 4

EP = 327680
NPAD = EP - E
NTRASH = 8
N2 = N + NTRASH

EROWS = EP // CHUNK
EPW = EP // NW
NCHUNK = EPW // CHUNK
NITER = NCHUNK // NBUF

EPT = EP // NS
NCHUNK0 = EPT // CHUNK
NITER0 = NCHUNK0 // NBUF

RPT = 624
RPT_LAST = N - RPT * (NS - 1)

_SC_PARAMS = pltpu.CompilerParams(use_tc_tiling_on_sc=False)


def _stripe_copy(src, dst, sid):
    r0 = sid * RPT

    @pl.when(sid < NS - 1)
    def _():
        pltpu.sync_copy(src.at[pl.ds(r0, RPT)], dst.at[pl.ds(r0, RPT)])

    @pl.when(sid == NS - 1)
    def _():
        pltpu.sync_copy(src.at[pl.ds(RPT * (NS - 1), RPT_LAST)],
                        dst.at[pl.ds(RPT * (NS - 1), RPT_LAST)])


def _edge_loop(h_ref, sidx, didx, rows, acc, gsem, ssem, niter):
    for b in range(NBUF):
        pltpu.async_copy(h_ref.at[sidx.at[b]], rows[b], gsem[b])

    def body(j, _):
        k0 = j * NBUF
        for b in range(NBUF):
            pltpu.make_async_copy(h_ref.at[sidx.at[k0 + b]], rows[b],
                                  gsem[b]).wait()
            pltpu.async_copy(rows[b], acc.at[didx.at[k0 + b]], ssem[b],
                             add=True)
        for b in range(NBUF):
            pltpu.make_async_copy(rows[b], acc.at[didx.at[k0 + b]],
                                  ssem[b]).wait()

            @pl.when(j < niter - 1)
            def _():
                pltpu.async_copy(h_ref.at[sidx.at[k0 + NBUF + b]], rows[b],
                                 gsem[b])
        return ()

    lax.fori_loop(0, niter, body, ())


@functools.lru_cache(maxsize=None)
def _make_aggr0():
    mesh = plsc.VectorSubcoreMesh(core_axis_name="c", subcore_axis_name="s")

    @functools.partial(
        pl.kernel,
        out_type=jax.ShapeDtypeStruct((NC, N, H), jnp.float32),
        mesh=mesh,
        scratch_types=[
            pltpu.VMEM((NCHUNK0, CHUNK), jnp.int32),
            pltpu.VMEM((NCHUNK0, CHUNK), jnp.int32),
            [pltpu.VMEM((CHUNK, H), jnp.float32) for _ in range(NBUF)],
            pltpu.VMEM_SHARED((N2, H), jnp.float32),
            [pltpu.SemaphoreType.DMA for _ in range(NBUF)],
            [pltpu.SemaphoreType.DMA for _ in range(NBUF)],
        ],
        compiler_params=_SC_PARAMS,
    )
    def aggr0(hlo_hbm, hhi_hbm, src_hbm, dst_hbm, out_hbm, sidx, didx, rows,
              acc, gsem, ssem):
        cid = lax.axis_index("c")
        sid = lax.axis_index("s")
        pltpu.sync_copy(src_hbm.at[pl.ds(sid * NCHUNK0, NCHUNK0)], sidx)
        pltpu.sync_copy(dst_hbm.at[pl.ds(sid * NCHUNK0, NCHUNK0)], didx)

        @pl.when(cid == 0)
        def _():
            _stripe_copy(hlo_hbm, acc, sid)

        @pl.when(cid == 1)
        def _():
            _stripe_copy(hhi_hbm, acc, sid)

        plsc.subcore_barrier()

        @pl.when(cid == 0)
        def _():
            _edge_loop(hlo_hbm, sidx, didx, rows, acc, gsem, ssem, NITER0)

        @pl.when(cid == 1)
        def _():
            _edge_loop(hhi_hbm, sidx, didx, rows, acc, gsem, ssem, NITER0)

        plsc.subcore_barrier()
        _stripe_copy(acc, out_hbm.at[cid], sid)

    return aggr0


@functools.lru_cache(maxsize=None)
def _make_aggr():
    mesh = plsc.VectorSubcoreMesh(core_axis_name="c", subcore_axis_name="s")

    @functools.partial(
        pl.kernel,
        out_type=jax.ShapeDtypeStruct((NC, N, H), jnp.float32),
        mesh=mesh,
        scratch_types=[
            pltpu.VMEM((NCHUNK, CHUNK), jnp.int32),
            pltpu.VMEM((NCHUNK, CHUNK), jnp.int32),
            [pltpu.VMEM((CHUNK, H), jnp.float32) for _ in range(NBUF)],
            pltpu.VMEM_SHARED((N2, H), jnp.float32),
            [pltpu.SemaphoreType.DMA for _ in range(NBUF)],
            [pltpu.SemaphoreType.DMA for _ in range(NBUF)],
        ],
        compiler_params=_SC_PARAMS,
    )
    def aggr(h_hbm, src_hbm, dst_hbm, out_hbm, sidx, didx, rows,
             acc, gsem, ssem):
        cid = lax.axis_index("c")
        sid = lax.axis_index("s")
        wid = cid * NS + sid
        pltpu.sync_copy(src_hbm.at[pl.ds(wid * NCHUNK, NCHUNK)], sidx)
        pltpu.sync_copy(dst_hbm.at[pl.ds(wid * NCHUNK, NCHUNK)], didx)
        _stripe_copy(h_hbm, acc, sid)
        plsc.subcore_barrier()
        _edge_loop(h_hbm, sidx, didx, rows, acc, gsem, ssem, NITER)
        plsc.subcore_barrier()
        _stripe_copy(acc, out_hbm.at[cid], sid)

    return aggr


NP = N // 2
_BN = 1000
_NBLK = NP // _BN


def _mlp_halves(acc_lo_e, acc_lo_o, acc_hi_e, acc_hi_o,
                w1_ref, b1_ref, w2_ref, b2_ref):
    w1lo = w1_ref[:H]
    w1hi = w1_ref[H:]
    outs = []
    for alo, ahi in ((acc_lo_e, acc_hi_e), (acc_lo_o, acc_hi_o)):
        t = (jnp.dot(alo, w1lo, preferred_element_type=jnp.float32)
             + jnp.dot(ahi, w1hi, preferred_element_type=jnp.float32))
        t = jnp.maximum(t + b1_ref[...], 0.0)
        u = jnp.dot(t, w2_ref[...], preferred_element_type=jnp.float32)
        outs.append(jnp.maximum(u + b2_ref[...], 0.0))
    return outs


def _mlp0_body(a_ref, w1_ref, b1_ref, w2_ref, b2_ref, o_ref):
    lo = a_ref[0]
    hi = a_ref[1]
    he, ho = _mlp_halves(lo[:, :H], lo[:, H:], hi[:, :H], hi[:, H:],
                         w1_ref, b1_ref, w2_ref, b2_ref)
    o_ref[...] = jnp.concatenate([he, ho], axis=1)


_MLP0 = pl.pallas_call(
    _mlp0_body,
    grid=(_NBLK,),
    in_specs=[
        pl.BlockSpec((NC, _BN, D0), lambda i: (0, i, 0)),
        pl.BlockSpec((D0, H), lambda i: (0, 0)),
        pl.BlockSpec((1, H), lambda i: (0, 0)),
        pl.BlockSpec((H, H), lambda i: (0, 0)),
        pl.BlockSpec((1, H), lambda i: (0, 0)),
    ],
    out_specs=pl.BlockSpec((_BN, D0), lambda i: (i, 0)),
    out_shape=jax.ShapeDtypeStruct((NP, D0), jnp.float32),
)


def _mlp_halves64(acc, w1_ref, b1_ref, w2_ref, b2_ref):
    outs = []
    for a in (acc[:, :H], acc[:, H:]):
        t = jnp.dot(a, w1_ref[...], preferred_element_type=jnp.float32)
        t = jnp.maximum(t + b1_ref[...], 0.0)
        u = jnp.dot(t, w2_ref[...], preferred_element_type=jnp.float32)
        outs.append(jnp.maximum(u + b2_ref[...], 0.0))
    return outs


def _mlp_body(h_ref, a_ref, w1_ref, b1_ref, w2_ref, b2_ref, o_ref):
    acc = a_ref[0] + a_ref[1] - h_ref[...]
    he, ho = _mlp_halves64(acc, w1_ref, b1_ref, w2_ref, b2_ref)
    o_ref[...] = jnp.concatenate([he, ho], axis=1)


_MLP = pl.pallas_call(
    _mlp_body,
    grid=(_NBLK,),
    in_specs=[
        pl.BlockSpec((_BN, D0), lambda i: (i, 0)),
        pl.BlockSpec((NC, _BN, D0), lambda i: (0, i, 0)),
        pl.BlockSpec((H, H), lambda i: (0, 0)),
        pl.BlockSpec((1, H), lambda i: (0, 0)),
        pl.BlockSpec((H, H), lambda i: (0, 0)),
        pl.BlockSpec((1, H), lambda i: (0, 0)),
    ],
    out_specs=pl.BlockSpec((_BN, D0), lambda i: (i, 0)),
    out_shape=jax.ShapeDtypeStruct((NP, D0), jnp.float32),
)


def _final_body(h_ref, a_ref, be_ref, bo_ref, w1_ref, b1_ref, w2_ref, b2_ref,
                l1w_ref, l1b_ref, l2w_ref, l2b_ref, o_ref, sums, counts):
    i = pl.program_id(0)

    @pl.when(i == 0)
    def _():
        sums[...] = jnp.zeros_like(sums)
        counts[...] = jnp.zeros_like(counts)

    acc = a_ref[0] + a_ref[1] - h_ref[...]
    h3e, h3o = _mlp_halves64(acc, w1_ref, b1_ref, w2_ref, b2_ref)

    ones = jnp.ones((_BN, 1), jnp.float32)
    for b_ref, h3 in ((be_ref, h3e), (bo_ref, h3o)):
        b = b_ref[0, 0, :]
        onehot = (b[:, None] == lax.broadcasted_iota(jnp.int32, (_BN, G), 1))
        onehot = onehot.astype(jnp.float32)
        sums[...] += lax.dot_general(onehot, h3, (((0,), (0,)), ((), ())),
                                     preferred_element_type=jnp.float32)
        counts[...] += lax.dot_general(onehot, ones, (((0,), (0,)), ((), ())),
                                       preferred_element_type=jnp.float32)

    @pl.when(i == _NBLK - 1)
    def _():
        pooled = sums[...] / jnp.maximum(counts[...], 1.0)
        y = jnp.dot(pooled, l1w_ref[...], preferred_element_type=jnp.float32)
        y = jnp.maximum(y + l1b_ref[...], 0.0)
        o_ref[...] = (jnp.dot(y, l2w_ref[...],
                              preferred_element_type=jnp.float32)
                      + l2b_ref[...])


_FINAL = pl.pallas_call(
    _final_body,
    grid=(_NBLK,),
    in_specs=[
        pl.BlockSpec((_BN, D0), lambda i: (i, 0)),
        pl.BlockSpec((NC, _BN, D0), lambda i: (0, i, 0)),
        pl.BlockSpec((1, 1, _BN), lambda i: (i, 0, 0)),
        pl.BlockSpec((1, 1, _BN), lambda i: (i, 0, 0)),
        pl.BlockSpec((H, H), lambda i: (0, 0)),
        pl.BlockSpec((1, H), lambda i: (0, 0)),
        pl.BlockSpec((H, H), lambda i: (0, 0)),
        pl.BlockSpec((1, H), lambda i: (0, 0)),
        pl.BlockSpec((H, H), lambda i: (0, 0)),
        pl.BlockSpec((1, H), lambda i: (0, 0)),
        pl.BlockSpec((H, C), lambda i: (0, 0)),
        pl.BlockSpec((1, C), lambda i: (0, 0)),
    ],
    out_specs=pl.BlockSpec((G, C), lambda i: (0, 0)),
    out_shape=jax.ShapeDtypeStruct((G, C), jnp.float32),
    scratch_shapes=[
        pltpu.VMEM((G, H), jnp.float32),
        pltpu.VMEM((G, 1), jnp.float32),
    ],
)


def kernel(x, edge_index, batch,
           gin0_W1, gin0_b1, gin0_W2, gin0_b2,
           gin1_W1, gin1_b1, gin1_W2, gin1_b2,
           gin2_W1, gin2_b1, gin2_W2, gin2_b2,
           lin1_W, lin1_b, lin2_W, lin2_b):
    pad_i = jnp.arange(NPAD, dtype=jnp.int32)
    src = jnp.concatenate([edge_index[0].astype(jnp.int32),
                           pad_i % N]).reshape(EROWS, CHUNK)
    dst = jnp.concatenate([edge_index[1].astype(jnp.int32),
                           N + (pad_i % NTRASH)]).reshape(EROWS, CHUNK)
    bi = batch.astype(jnp.int32).reshape(NP, 2)
    be = bi[:, 0].reshape(_NBLK, 1, _BN)
    bo = bi[:, 1].reshape(_NBLK, 1, _BN)

    a = _make_aggr0()(x[:, :H], x[:, H:], src, dst)
    h1 = _MLP0(a.reshape(NC, NP, D0), gin0_W1, gin0_b1.reshape(1, H),
               gin0_W2, gin0_b2.reshape(1, H))
    a = _make_aggr()(h1.reshape(N, H), src, dst)
    h2 = _MLP(h1, a.reshape(NC, NP, D0), gin1_W1, gin1_b1.reshape(1, H),
              gin1_W2, gin1_b2.reshape(1, H))
    a = _make_aggr()(h2.reshape(N, H), src, dst)
    out = _FINAL(h2, a.reshape(NC, NP, D0), be, bo,
                 gin2_W1, gin2_b1.reshape(1, H),
                 gin2_W2, gin2_b2.reshape(1, H),
                 lin1_W, lin1_b.reshape(1, H),
                 lin2_W, lin2_b.reshape(1, C))
    return out


# speedup vs baseline: 16.7666x; 1.0366x over previous
import functools

import jax
import jax.numpy as jnp
from jax import lax
from jax.experimental import pallas as pl
from jax.experimental.pallas import tpu as pltpu
from jax.experimental.pallas import tpu_sc as plsc

N = 10000
E = 320000
D0 = 128
H = 64
C = 2
G = 64

NC = 2
NS = 16
NW = NC * NS
CHUNK = 128
NBUF0 = 5
NBUF1 = 8

EP = 327680
NPAD = EP - E
NTRASH = 8
N2 = N + NTRASH

EROWS = EP // CHUNK
EPW = EP // NW
NCHUNK = EPW // CHUNK
NITER = NCHUNK // NBUF1

EPT = EP // NS
NCHUNK0 = EPT // CHUNK
NITER0 = NCHUNK0 // NBUF0

RPT = 624
RPT_LAST = N - RPT * (NS - 1)

_SC_PARAMS = pltpu.CompilerParams(use_tc_tiling_on_sc=False)


def _stripe_copy(src, dst, sid):
    r0 = sid * RPT

    @pl.when(sid < NS - 1)
    def _():
        pltpu.sync_copy(src.at[pl.ds(r0, RPT)], dst.at[pl.ds(r0, RPT)])

    @pl.when(sid == NS - 1)
    def _():
        pltpu.sync_copy(src.at[pl.ds(RPT * (NS - 1), RPT_LAST)],
                        dst.at[pl.ds(RPT * (NS - 1), RPT_LAST)])


def _edge_loop(h_ref, sidx, didx, rows, acc, gsem, ssem, niter):
    nbuf = len(rows)
    for b in range(nbuf):
        pltpu.async_copy(h_ref.at[sidx.at[b]], rows[b], gsem[b])

    def body(j, _):
        k0 = j * nbuf
        for b in range(nbuf):
            pltpu.make_async_copy(h_ref.at[sidx.at[k0 + b]], rows[b],
                                  gsem[b]).wait()
            pltpu.async_copy(rows[b], acc.at[didx.at[k0 + b]], ssem[b],
                             add=True)
        for b in range(nbuf):
            pltpu.make_async_copy(rows[b], acc.at[didx.at[k0 + b]],
                                  ssem[b]).wait()

            @pl.when(j < niter - 1)
            def _():
                pltpu.async_copy(h_ref.at[sidx.at[k0 + nbuf + b]], rows[b],
                                 gsem[b])
        return ()

    lax.fori_loop(0, niter, body, ())


@functools.lru_cache(maxsize=None)
def _make_aggr0():
    mesh = plsc.VectorSubcoreMesh(core_axis_name="c", subcore_axis_name="s")

    @functools.partial(
        pl.kernel,
        out_type=jax.ShapeDtypeStruct((NC, N, H), jnp.float32),
        mesh=mesh,
        scratch_types=[
            pltpu.VMEM((NCHUNK0, CHUNK), jnp.int32),
            pltpu.VMEM((NCHUNK0, CHUNK), jnp.int32),
            [pltpu.VMEM((CHUNK, H), jnp.float32) for _ in range(NBUF0)],
            pltpu.VMEM_SHARED((N2, H), jnp.float32),
            [pltpu.SemaphoreType.DMA for _ in range(NBUF0)],
            [pltpu.SemaphoreType.DMA for _ in range(NBUF0)],
        ],
        compiler_params=_SC_PARAMS,
    )
    def aggr0(hlo_hbm, hhi_hbm, src_hbm, dst_hbm, out_hbm, sidx, didx, rows,
              acc, gsem, ssem):
        cid = lax.axis_index("c")
        sid = lax.axis_index("s")
        pltpu.sync_copy(src_hbm.at[pl.ds(sid * NCHUNK0, NCHUNK0)], sidx)
        pltpu.sync_copy(dst_hbm.at[pl.ds(sid * NCHUNK0, NCHUNK0)], didx)

        @pl.when(cid == 0)
        def _():
            _stripe_copy(hlo_hbm, acc, sid)

        @pl.when(cid == 1)
        def _():
            _stripe_copy(hhi_hbm, acc, sid)

        plsc.subcore_barrier()

        @pl.when(cid == 0)
        def _():
            _edge_loop(hlo_hbm, sidx, didx, rows, acc, gsem, ssem, NITER0)

        @pl.when(cid == 1)
        def _():
            _edge_loop(hhi_hbm, sidx, didx, rows, acc, gsem, ssem, NITER0)

        plsc.subcore_barrier()
        _stripe_copy(acc, out_hbm.at[cid], sid)

    return aggr0


@functools.lru_cache(maxsize=None)
def _make_aggr():
    mesh = plsc.VectorSubcoreMesh(core_axis_name="c", subcore_axis_name="s")

    @functools.partial(
        pl.kernel,
        out_type=jax.ShapeDtypeStruct((NC, N, H), jnp.float32),
        mesh=mesh,
        scratch_types=[
            pltpu.VMEM((NCHUNK, CHUNK), jnp.int32),
            pltpu.VMEM((NCHUNK, CHUNK), jnp.int32),
            [pltpu.VMEM((CHUNK, H), jnp.float32) for _ in range(NBUF1)],
            pltpu.VMEM_SHARED((N2, H), jnp.float32),
            [pltpu.SemaphoreType.DMA for _ in range(NBUF1)],
            [pltpu.SemaphoreType.DMA for _ in range(NBUF1)],
        ],
        compiler_params=_SC_PARAMS,
    )
    def aggr(h_hbm, src_hbm, dst_hbm, out_hbm, sidx, didx, rows,
             acc, gsem, ssem):
        cid = lax.axis_index("c")
        sid = lax.axis_index("s")
        wid = cid * NS + sid
        pltpu.sync_copy(src_hbm.at[pl.ds(wid * NCHUNK, NCHUNK)], sidx)
        pltpu.sync_copy(dst_hbm.at[pl.ds(wid * NCHUNK, NCHUNK)], didx)
        _stripe_copy(h_hbm, acc, sid)
        plsc.subcore_barrier()
        _edge_loop(h_hbm, sidx, didx, rows, acc, gsem, ssem, NITER)
        plsc.subcore_barrier()
        _stripe_copy(acc, out_hbm.at[cid], sid)

    return aggr


NP = N // 2
_BN = 1000
_NBLK = NP // _BN


def _mlp_halves(acc_lo_e, acc_lo_o, acc_hi_e, acc_hi_o,
                w1_ref, b1_ref, w2_ref, b2_ref):
    w1lo = w1_ref[:H]
    w1hi = w1_ref[H:]
    outs = []
    for alo, ahi in ((acc_lo_e, acc_hi_e), (acc_lo_o, acc_hi_o)):
        t = (jnp.dot(alo, w1lo, preferred_element_type=jnp.float32)
             + jnp.dot(ahi, w1hi, preferred_element_type=jnp.float32))
        t = jnp.maximum(t + b1_ref[...], 0.0)
        u = jnp.dot(t, w2_ref[...], preferred_element_type=jnp.float32)
        outs.append(jnp.maximum(u + b2_ref[...], 0.0))
    return outs


def _mlp0_body(a_ref, w1_ref, b1_ref, w2_ref, b2_ref, o_ref):
    lo = a_ref[0]
    hi = a_ref[1]
    he, ho = _mlp_halves(lo[:, :H], lo[:, H:], hi[:, :H], hi[:, H:],
                         w1_ref, b1_ref, w2_ref, b2_ref)
    o_ref[...] = jnp.concatenate([he, ho], axis=1)


_MLP0 = pl.pallas_call(
    _mlp0_body,
    grid=(_NBLK,),
    in_specs=[
        pl.BlockSpec((NC, _BN, D0), lambda i: (0, i, 0)),
        pl.BlockSpec((D0, H), lambda i: (0, 0)),
        pl.BlockSpec((1, H), lambda i: (0, 0)),
        pl.BlockSpec((H, H), lambda i: (0, 0)),
        pl.BlockSpec((1, H), lambda i: (0, 0)),
    ],
    out_specs=pl.BlockSpec((_BN, D0), lambda i: (i, 0)),
    out_shape=jax.ShapeDtypeStruct((NP, D0), jnp.float32),
)


def _mlp_halves64(acc, w1_ref, b1_ref, w2_ref, b2_ref):
    outs = []
    for a in (acc[:, :H], acc[:, H:]):
        t = jnp.dot(a, w1_ref[...], preferred_element_type=jnp.float32)
        t = jnp.maximum(t + b1_ref[...], 0.0)
        u = jnp.dot(t, w2_ref[...], preferred_element_type=jnp.float32)
        outs.append(jnp.maximum(u + b2_ref[...], 0.0))
    return outs


def _mlp_body(h_ref, a_ref, w1_ref, b1_ref, w2_ref, b2_ref, o_ref):
    acc = a_ref[0] + a_ref[1] - h_ref[...]
    he, ho = _mlp_halves64(acc, w1_ref, b1_ref, w2_ref, b2_ref)
    o_ref[...] = jnp.concatenate([he, ho], axis=1)


_MLP = pl.pallas_call(
    _mlp_body,
    grid=(_NBLK,),
    in_specs=[
        pl.BlockSpec((_BN, D0), lambda i: (i, 0)),
        pl.BlockSpec((NC, _BN, D0), lambda i: (0, i, 0)),
        pl.BlockSpec((H, H), lambda i: (0, 0)),
        pl.BlockSpec((1, H), lambda i: (0, 0)),
        pl.BlockSpec((H, H), lambda i: (0, 0)),
        pl.BlockSpec((1, H), lambda i: (0, 0)),
    ],
    out_specs=pl.BlockSpec((_BN, D0), lambda i: (i, 0)),
    out_shape=jax.ShapeDtypeStruct((NP, D0), jnp.float32),
)


def _final_body(h_ref, a_ref, be_ref, bo_ref, w1_ref, b1_ref, w2_ref, b2_ref,
                l1w_ref, l1b_ref, l2w_ref, l2b_ref, o_ref, sums, counts):
    i = pl.program_id(0)

    @pl.when(i == 0)
    def _():
        sums[...] = jnp.zeros_like(sums)
        counts[...] = jnp.zeros_like(counts)

    acc = a_ref[0] + a_ref[1] - h_ref[...]
    h3e, h3o = _mlp_halves64(acc, w1_ref, b1_ref, w2_ref, b2_ref)

    ones = jnp.ones((_BN, 1), jnp.float32)
    for b_ref, h3 in ((be_ref, h3e), (bo_ref, h3o)):
        b = b_ref[0, 0, :]
        onehot = (b[:, None] == lax.broadcasted_iota(jnp.int32, (_BN, G), 1))
        onehot = onehot.astype(jnp.float32)
        sums[...] += lax.dot_general(onehot, h3, (((0,), (0,)), ((), ())),
                                     preferred_element_type=jnp.float32)
        counts[...] += lax.dot_general(onehot, ones, (((0,), (0,)), ((), ())),
                                       preferred_element_type=jnp.float32)

    @pl.when(i == _NBLK - 1)
    def _():
        pooled = sums[...] / jnp.maximum(counts[...], 1.0)
        y = jnp.dot(pooled, l1w_ref[...], preferred_element_type=jnp.float32)
        y = jnp.maximum(y + l1b_ref[...], 0.0)
        o_ref[...] = (jnp.dot(y, l2w_ref[...],
                              preferred_element_type=jnp.float32)
                      + l2b_ref[...])


_FINAL = pl.pallas_call(
    _final_body,
    grid=(_NBLK,),
    in_specs=[
        pl.BlockSpec((_BN, D0), lambda i: (i, 0)),
        pl.BlockSpec((NC, _BN, D0), lambda i: (0, i, 0)),
        pl.BlockSpec((1, 1, _BN), lambda i: (i, 0, 0)),
        pl.BlockSpec((1, 1, _BN), lambda i: (i, 0, 0)),
        pl.BlockSpec((H, H), lambda i: (0, 0)),
        pl.BlockSpec((1, H), lambda i: (0, 0)),
        pl.BlockSpec((H, H), lambda i: (0, 0)),
        pl.BlockSpec((1, H), lambda i: (0, 0)),
        pl.BlockSpec((H, H), lambda i: (0, 0)),
        pl.BlockSpec((1, H), lambda i: (0, 0)),
        pl.BlockSpec((H, C), lambda i: (0, 0)),
        pl.BlockSpec((1, C), lambda i: (0, 0)),
    ],
    out_specs=pl.BlockSpec((G, C), lambda i: (0, 0)),
    out_shape=jax.ShapeDtypeStruct((G, C), jnp.float32),
    scratch_shapes=[
        pltpu.VMEM((G, H), jnp.float32),
        pltpu.VMEM((G, 1), jnp.float32),
    ],
)


def kernel(x, edge_index, batch,
           gin0_W1, gin0_b1, gin0_W2, gin0_b2,
           gin1_W1, gin1_b1, gin1_W2, gin1_b2,
           gin2_W1, gin2_b1, gin2_W2, gin2_b2,
           lin1_W, lin1_b, lin2_W, lin2_b):
    pad_i = jnp.arange(NPAD, dtype=jnp.int32)
    src = jnp.concatenate([edge_index[0].astype(jnp.int32),
                           pad_i % N]).reshape(EROWS, CHUNK)
    dst = jnp.concatenate([edge_index[1].astype(jnp.int32),
                           N + (pad_i % NTRASH)]).reshape(EROWS, CHUNK)
    bi = batch.astype(jnp.int32).reshape(NP, 2)
    be = bi[:, 0].reshape(_NBLK, 1, _BN)
    bo = bi[:, 1].reshape(_NBLK, 1, _BN)

    a = _make_aggr0()(x[:, :H], x[:, H:], src, dst)
    h1 = _MLP0(a.reshape(NC, NP, D0), gin0_W1, gin0_b1.reshape(1, H),
               gin0_W2, gin0_b2.reshape(1, H))
    a = _make_aggr()(h1.reshape(N, H), src, dst)
    h2 = _MLP(h1, a.reshape(NC, NP, D0), gin1_W1, gin1_b1.reshape(1, H),
              gin1_W2, gin1_b2.reshape(1, H))
    a = _make_aggr()(h2.reshape(N, H), src, dst)
    out = _FINAL(h2, a.reshape(NC, NP, D0), be, bo,
                 gin2_W1, gin2_b1.reshape(1, H),
                 gin2_W2, gin2_b2.reshape(1, H),
                 lin1_W, lin1_b.reshape(1, H),
                 lin2_W, lin2_b.reshape(1, C))
    return out
